# baseline (speedup 1.0000x reference)
import jax
import jax.numpy as jnp
from jax.experimental import pallas as pl
from jax.experimental.pallas import tpu as pltpu

_ALPHA = 0.5
_B, _T = 8192, 64
_IB = 4096
_NI = _B // _IB
_JB = 1024
_NJ = _B // _JB


def _deephit_block(pblk_ref, pt_ref, trow_ref, tcol_ref, ecol_ref,
                   rank_ref, cnt_ref, lik_ref, ptm_ref, ngt_ref):
    t_col = tcol_ref[0]
    e_col = ecol_ref[0]
    p_blk = pblk_ref[...]

    @pl.when(pl.program_id(1) == 0)
    def _prep():
        sub_t = jax.lax.broadcasted_iota(jnp.int32, (_T + 8, _B), 0)
        cmpf = jnp.where(trow_ref[...] > sub_t, 1.0, 0.0).astype(jnp.float32)
        onesrow = jnp.where(sub_t == _T, 1.0, 0.0).astype(jnp.float32)
        ptm_ref[...] = (pt_ref[...] * cmpf + onesrow).astype(jnp.bfloat16)
        ngt_ref[...] = jnp.sum(cmpf, axis=1, keepdims=True)

    lane_t = jax.lax.broadcasted_iota(jnp.int32, (_IB, _T), 1)
    onehot = lane_t == t_col
    p_i = jnp.sum(jnp.where(onehot, p_blk, 0.0), axis=1, keepdims=True)

    lane_a = jax.lax.broadcasted_iota(jnp.int32, (_IB, _T + 8), 1)
    e1h_augf = (jnp.where(lane_a == t_col, 1.0, 0.0)
                + jnp.where(lane_a == _T, 1.0, 0.0) * (-p_i))
    e1h_aug = e1h_augf.astype(jnp.bfloat16)
    c_col = jnp.dot(e1h_augf, ngt_ref[...], preferred_element_type=jnp.float32)

    s_acc = jnp.zeros((_IB, _JB), jnp.float32)
    for c in range(_NJ):
        d = jnp.dot(e1h_aug, ptm_ref[:, c * _JB:(c + 1) * _JB],
                    preferred_element_type=jnp.float32)
        s_acc = s_acc + jnp.maximum(d, 0.0)

    s_col = jnp.sum(s_acc, axis=1, keepdims=True)
    p_i_cl = jnp.clip(p_i, 1e-12, 1.0 - 1e-12)
    nll = -jnp.log(p_i_cl)

    rank_ref[...] = jnp.broadcast_to(jnp.sum(s_col * e_col), (1, 1, 1))
    cnt_ref[...] = jnp.broadcast_to(jnp.sum(c_col * e_col), (1, 1, 1))
    lik_ref[...] = jnp.broadcast_to(jnp.sum(nll * e_col), (1, 1, 1))


def kernel(preds, durations, events):
    t_idx = jnp.clip(durations.astype(jnp.int32) - 1, 0, _T - 1)
    pt = jnp.pad(preds.T, ((0, 8), (0, 0)))
    trow = t_idx.reshape(1, _B)
    tcol = t_idx.reshape(_NI, _IB, 1)
    ecol = events.astype(jnp.float32).reshape(_NI, _IB, 1)

    out_sds = jax.ShapeDtypeStruct((_NI, 1, 1), jnp.float32)
    blk = lambda o, i: (o * (_NI // 2) + i, 0, 0)
    rank_p, cnt_p, lik_p = pl.pallas_call(
        _deephit_block,
        grid=(2, _NI // 2),
        in_specs=[
            pl.BlockSpec((_IB, _T), lambda o, i: (o * (_NI // 2) + i, 0)),
            pl.BlockSpec((_T + 8, _B), lambda o, i: (0, 0)),
            pl.BlockSpec((1, _B), lambda o, i: (0, 0)),
            pl.BlockSpec((1, _IB, 1), blk),
            pl.BlockSpec((1, _IB, 1), blk),
        ],
        out_specs=[
            pl.BlockSpec((1, 1, 1), blk),
            pl.BlockSpec((1, 1, 1), blk),
            pl.BlockSpec((1, 1, 1), blk),
        ],
        out_shape=[out_sds, out_sds, out_sds],
        scratch_shapes=[
            pltpu.VMEM((_T + 8, _B), jnp.bfloat16),
            pltpu.VMEM((_T + 8, 1), jnp.float32),
        ],
        compiler_params=pltpu.CompilerParams(
            dimension_semantics=("parallel", "arbitrary"),
        ),
        name="deephit_loss",
    )(preds, pt, trow, tcol, ecol)

    rank_tot = jnp.sum(rank_p)
    cnt_tot = jnp.sum(cnt_p)
    lik_tot = jnp.sum(lik_p)
    rank = jnp.where(cnt_tot > 0, rank_tot / cnt_tot, jnp.float32(0.0))
    return _ALPHA * (lik_tot / _B) + (1.0 - _ALPHA) * rank


# speedup vs baseline: 1.0156x; 1.0156x over previous
import jax
import jax.numpy as jnp
from jax.experimental import pallas as pl
from jax.experimental.pallas import tpu as pltpu

_ALPHA = 0.5
_B, _T = 8192, 64
_IB = 4096
_NI = _B // _IB
_JB = 256
_NJ = _B // _JB


def _deephit_block(pblk_ref, pt_ref, trow_ref, tcol_ref, ecol_ref,
                   rank_ref, cnt_ref, lik_ref, ptm_ref, ngt_ref):
    t_col = tcol_ref[0]
    e_col = ecol_ref[0]
    p_blk = pblk_ref[...]

    @pl.when(pl.program_id(1) == 0)
    def _prep():
        sub_t = jax.lax.broadcasted_iota(jnp.int32, (_T + 8, _B), 0)
        cmpf = jnp.where(trow_ref[...] > sub_t, 1.0, 0.0).astype(jnp.float32)
        onesrow = jnp.where(sub_t == _T, 1.0, 0.0).astype(jnp.float32)
        ptm_ref[...] = (pt_ref[...] * cmpf + onesrow).astype(jnp.bfloat16)
        ngt_ref[...] = jnp.sum(cmpf, axis=1, keepdims=True)

    lane_t = jax.lax.broadcasted_iota(jnp.int32, (_IB, _T), 1)
    onehot = lane_t == t_col
    p_i = jnp.sum(jnp.where(onehot, p_blk, 0.0), axis=1, keepdims=True)

    lane_a = jax.lax.broadcasted_iota(jnp.int32, (_IB, _T + 8), 1)
    e1h_augf = (jnp.where(lane_a == t_col, 1.0, 0.0)
                + jnp.where(lane_a == _T, 1.0, 0.0) * (-p_i))
    e1h_aug = e1h_augf.astype(jnp.bfloat16)
    c_col = jnp.dot(e1h_augf, ngt_ref[...], preferred_element_type=jnp.float32)

    s_acc = jnp.zeros((_IB, _JB), jnp.float32)
    for c in range(_NJ):
        d = jnp.dot(e1h_aug, ptm_ref[:, c * _JB:(c + 1) * _JB],
                    preferred_element_type=jnp.float32)
        s_acc = s_acc + jnp.maximum(d, 0.0)

    s_col = jnp.sum(s_acc, axis=1, keepdims=True)
    p_i_cl = jnp.clip(p_i, 1e-12, 1.0 - 1e-12)
    nll = -jnp.log(p_i_cl)

    rank_ref[...] = jnp.broadcast_to(jnp.sum(s_col * e_col), (1, 1, 1))
    cnt_ref[...] = jnp.broadcast_to(jnp.sum(c_col * e_col), (1, 1, 1))
    lik_ref[...] = jnp.broadcast_to(jnp.sum(nll * e_col), (1, 1, 1))


def kernel(preds, durations, events):
    t_idx = jnp.clip(durations.astype(jnp.int32) - 1, 0, _T - 1)
    pt = jnp.pad(preds.T, ((0, 8), (0, 0)))
    trow = t_idx.reshape(1, _B)
    tcol = t_idx.reshape(_NI, _IB, 1)
    ecol = events.astype(jnp.float32).reshape(_NI, _IB, 1)

    out_sds = jax.ShapeDtypeStruct((_NI, 1, 1), jnp.float32)
    blk = lambda o, i: (o * (_NI // 2) + i, 0, 0)
    rank_p, cnt_p, lik_p = pl.pallas_call(
        _deephit_block,
        grid=(2, _NI // 2),
        in_specs=[
            pl.BlockSpec((_IB, _T), lambda o, i: (o * (_NI // 2) + i, 0)),
            pl.BlockSpec((_T + 8, _B), lambda o, i: (0, 0)),
            pl.BlockSpec((1, _B), lambda o, i: (0, 0)),
            pl.BlockSpec((1, _IB, 1), blk),
            pl.BlockSpec((1, _IB, 1), blk),
        ],
        out_specs=[
            pl.BlockSpec((1, 1, 1), blk),
            pl.BlockSpec((1, 1, 1), blk),
            pl.BlockSpec((1, 1, 1), blk),
        ],
        out_shape=[out_sds, out_sds, out_sds],
        scratch_shapes=[
            pltpu.VMEM((_T + 8, _B), jnp.bfloat16),
            pltpu.VMEM((_T + 8, 1), jnp.float32),
        ],
        compiler_params=pltpu.CompilerParams(
            dimension_semantics=("parallel", "arbitrary"),
        ),
        name="deephit_loss",
    )(preds, pt, trow, tcol, ecol)

    rank_tot = jnp.sum(rank_p)
    cnt_tot = jnp.sum(cnt_p)
    lik_tot = jnp.sum(lik_p)
    rank = jnp.where(cnt_tot > 0, rank_tot / cnt_tot, jnp.float32(0.0))
    return _ALPHA * (lik_tot / _B) + (1.0 - _ALPHA) * rank
